# Initial kernel scaffold; baseline (speedup 1.0000x reference)
import jax
import jax.numpy as jnp
from jax.experimental import pallas as pl


def kernel(adj_UJ_indices, adj_UJ_values, adj_IJ_indices, adj_IJ_values, top_embs, pos_bottoms_embs, all_users_embs):
    raise NotImplementedError("write your pallas kernel here")


# speedup vs baseline: 2.2750x; 2.2750x over previous
import jax
import jax.numpy as jnp
from jax import lax
from jax.experimental import pallas as pl
from jax.experimental.pallas import tpu as pltpu
from jax.experimental.pallas import tpu_sc as plsc

N_ROWS = 10000
N_ACC = 10240
D = 256
DH = 128
E = 160000
NT = 16
C = 80
E_PAD = 163840
CPT = E_PAD // NT // C
RPT = N_ACC // NT
ZR = 128


def _sc_lightgcn(bot, usr, top, ujr, ujc, ujv, ijr, ijc, ijv):
    mesh = plsc.VectorSubcoreMesh(core_axis_name="c", subcore_axis_name="s")
    f32 = jnp.float32

    def body(bot_hbm, usr_hbm, top_hbm,
             ujr_hbm, ujc_hbm, ujv_hbm, ijr_hbm, ijc_hbm, ijv_hbm,
             out_u_hbm, out_t_hbm, out_p_hbm,
             acc, rows_v, cols_v, vals_v, gbuf):
        cid = lax.axis_index("c")
        tid = lax.axis_index("s")
        half_off = cid * N_ROWS

        z16 = jnp.zeros((16,), f32)

        def zero_acc():
            def zfill(r, carry):
                for c8 in range(DH // 16):
                    gbuf[r, pl.ds(c8 * 16, 16)] = z16
                return carry
            lax.fori_loop(0, C, zfill, 0)
            for k in range(RPT // C):
                pltpu.sync_copy(gbuf, acc.at[pl.ds(tid * RPT + k * C, C)])

        def accumulate(rows_hbm, cols_hbm, vals_hbm, table_hbm):
            base = tid * CPT
            pltpu.sync_copy(rows_hbm.at[pl.ds(base, CPT)], rows_v)
            pltpu.sync_copy(cols_hbm.at[pl.ds(base, CPT)], cols_v)
            pltpu.sync_copy(vals_hbm.at[pl.ds(base, CPT)], vals_v)

            off16 = jnp.full((16,), half_off, jnp.int32)

            def fix(i, carry):
                r = i // (C // 16)
                c = (i % (C // 16)) * 16
                cols_v[r, pl.ds(c, 16)] = cols_v[r, pl.ds(c, 16)] + off16
                return carry
            lax.fori_loop(0, CPT * (C // 16), fix, 0)

            def chunk(g, carry):
                pltpu.sync_copy(table_hbm.at[cols_v.at[g]], gbuf)

                def egroup(q, c2):
                    vv = vals_v[g, pl.ds(q * 16, 16)]
                    for lane in range(16):
                        v = vv[lane]
                        e = q * 16 + lane
                        for d8 in range(DH // 16):
                            sl = pl.ds(d8 * 16, 16)
                            gbuf[e, sl] = gbuf[e, sl] * v
                    return c2
                lax.fori_loop(0, C // 16, egroup, 0)
                pltpu.sync_copy(gbuf, acc.at[rows_v.at[g]], add=True)
                return carry
            lax.fori_loop(0, CPT, chunk, 0)

        def writeback(out_hbm):
            r0 = tid * RPT
            pltpu.sync_copy(acc.at[pl.ds(r0, RPT)],
                            out_hbm.at[pl.ds(cid * N_ACC + r0, RPT)])

        zero_acc()
        plsc.subcore_barrier()
        accumulate(ujr_hbm, ujc_hbm, ujv_hbm, bot_hbm)
        plsc.subcore_barrier()
        writeback(out_u_hbm)

        zero_acc()
        plsc.subcore_barrier()
        accumulate(ijr_hbm, ijc_hbm, ijv_hbm, bot_hbm)
        plsc.subcore_barrier()
        writeback(out_t_hbm)

        zero_acc()
        plsc.subcore_barrier()
        accumulate(ujc_hbm, ujr_hbm, ujv_hbm, usr_hbm)
        accumulate(ijc_hbm, ijr_hbm, ijv_hbm, top_hbm)
        plsc.subcore_barrier()
        writeback(out_p_hbm)

    out_sds = jax.ShapeDtypeStruct((2 * N_ACC, DH), f32)
    run = pl.kernel(
        body,
        out_type=(out_sds, out_sds, out_sds),
        mesh=mesh,
        compiler_params=pltpu.CompilerParams(use_tc_tiling_on_sc=False),
        scratch_types=(
            pltpu.VMEM_SHARED((N_ACC, DH), f32),
            pltpu.VMEM((CPT, C), jnp.int32),
            pltpu.VMEM((CPT, C), jnp.int32),
            pltpu.VMEM((CPT, C), f32),
            pltpu.VMEM((C, DH), f32),
        ),
    )
    return run(bot, usr, top, ujr, ujc, ujv, ijr, ijc, ijv)


def kernel(adj_UJ_indices, adj_UJ_values, adj_IJ_indices, adj_IJ_values,
           top_embs, pos_bottoms_embs, all_users_embs):
    i32 = jnp.int32

    def pad_idx(x):
        return jnp.pad(x.astype(i32), (0, E_PAD - E)).reshape(E_PAD // C, C)

    def pad_val(x):
        return jnp.pad(x, (0, E_PAD - E)).reshape(E_PAD // C, C)

    ujr = pad_idx(adj_UJ_indices[0])
    ujc = pad_idx(adj_UJ_indices[1])
    ijr = pad_idx(adj_IJ_indices[0])
    ijc = pad_idx(adj_IJ_indices[1])
    ujv = pad_val(adj_UJ_values)
    ijv = pad_val(adj_IJ_values)

    def stack_halves(x):
        return jnp.concatenate([x[:, :DH], x[:, DH:]], axis=0)

    bot = stack_halves(pos_bottoms_embs)
    usr = stack_halves(all_users_embs)
    top = stack_halves(top_embs)

    out_u, out_t, out_p = _sc_lightgcn(bot, usr, top, ujr, ujc, ujv,
                                       ijr, ijc, ijv)

    def unstack(o):
        return jnp.concatenate([o[:N_ROWS], o[N_ACC:N_ACC + N_ROWS]], axis=1)

    return (unstack(out_u), unstack(out_t), unstack(out_p))


# speedup vs baseline: 3.1724x; 1.3945x over previous
import jax
import jax.numpy as jnp
from jax import lax
from jax.experimental import pallas as pl
from jax.experimental.pallas import tpu as pltpu
from jax.experimental.pallas import tpu_sc as plsc

N_ROWS = 10000
N_ACC = 10240
D = 256
DH = 128
E = 160000
NT = 16
C = 64
CPT = 160
PH = 80
E_PAD = NT * CPT * C
RPT = N_ACC // NT


def _sc_lightgcn(bot, usr, top, ujr, ujc, ujv, ijr, ijc, ijv):
    mesh = plsc.VectorSubcoreMesh(core_axis_name="c", subcore_axis_name="s")
    f32 = jnp.float32

    def body(bot_hbm, usr_hbm, top_hbm,
             ujr_hbm, ujc_hbm, ujv_hbm, ijr_hbm, ijc_hbm, ijv_hbm,
             out_u_hbm, out_t_hbm, out_p_hbm,
             acc, rows_v, cols_v, vals_v, gb0, gb1, gb2,
             sg0, sg1, sg2, ss0, ss1, ss2):
        cid = lax.axis_index("c")
        tid = lax.axis_index("s")
        half_off = cid * N_ROWS
        gb = (gb0, gb1, gb2)
        sg = (sg0, sg1, sg2)
        ss = (ss0, ss1, ss2)

        z16 = jnp.zeros((16,), f32)

        def zero_acc():
            def zfill(r, carry):
                for c8 in range(DH // 16):
                    gb0[r, pl.ds(c8 * 16, 16)] = z16
                return carry
            lax.fori_loop(0, C, zfill, 0)
            for k in range(RPT // C):
                pltpu.sync_copy(gb0, acc.at[pl.ds(tid * RPT + k * C, C)])

        def accumulate(rows_hbm, cols_hbm, vals_hbm, table_hbm):
            def start_g(g, b):
                pltpu.async_copy(table_hbm.at[cols_v.at[g]], gb[b], sg[b])

            def wait_g(b):
                pltpu.make_async_copy(table_hbm.at[cols_v.at[0]], gb[b],
                                      sg[b]).wait()

            def start_s(g, b):
                pltpu.async_copy(gb[b], acc.at[rows_v.at[g]], ss[b], add=True)

            def wait_s(b):
                pltpu.make_async_copy(gb[b], acc.at[rows_v.at[0]],
                                      ss[b]).wait()

            dnums = lax.GatherDimensionNumbers(
                offset_dims=(), collapsed_slice_dims=(0,),
                start_index_map=(0,))

            def scale(g, b):
                buf = gb[b]

                def egroup(q, c2):
                    vv = vals_v[g, pl.ds(q * 16, 16)]

                    def lanes(lane, c3):
                        bidx = jnp.full((16,), lane, jnp.int32)
                        v16 = lax.gather(
                            vv, bidx[:, None], dnums, (1,),
                            mode=lax.GatherScatterMode.PROMISE_IN_BOUNDS)
                        e = q * 16 + lane
                        for d8 in range(DH // 16):
                            sl = pl.ds(d8 * 16, 16)
                            buf[e, sl] = buf[e, sl] * v16
                        return c3
                    return lax.fori_loop(0, 16, lanes, c2)
                lax.fori_loop(0, C // 16, egroup, 0)

            for h in range(2):
                base = tid * CPT + h * PH
                pltpu.sync_copy(rows_hbm.at[pl.ds(base, PH)], rows_v)
                pltpu.sync_copy(cols_hbm.at[pl.ds(base, PH)], cols_v)
                pltpu.sync_copy(vals_hbm.at[pl.ds(base, PH)], vals_v)

                off16 = jnp.full((16,), half_off, jnp.int32)

                def fix(i, carry):
                    r = i // (C // 16)
                    c = (i % (C // 16)) * 16
                    cols_v[r, pl.ds(c, 16)] = cols_v[r, pl.ds(c, 16)] + off16
                    return carry
                lax.fori_loop(0, PH * (C // 16), fix, 0)

                start_g(0, 0)
                start_g(1, 1)
                wait_g(0)
                scale(0, 0)
                start_s(0, 0)
                start_g(2, 2)
                wait_g(1)
                scale(1, 1)
                start_s(1, 1)
                wait_s(0)
                start_g(3, 0)

                def steady(i, carry):
                    g3 = 2 + 3 * i
                    for j in range(3):
                        g = g3 + j
                        b = (2 + j) % 3
                        wait_g(b)
                        scale(g, b)
                        start_s(g, b)
                        wait_s((b + 2) % 3)
                        start_g(g + 2, (b + 2) % 3)
                    return carry
                lax.fori_loop(0, (PH - 5) // 3, steady, 0)

                wait_g(2)
                scale(PH - 3, 2)
                start_s(PH - 3, 2)
                wait_s(1)
                start_g(PH - 1, 1)

                wait_g(0)
                scale(PH - 2, 0)
                start_s(PH - 2, 0)
                wait_s(2)

                wait_g(1)
                scale(PH - 1, 1)
                start_s(PH - 1, 1)
                wait_s(0)
                wait_s(1)

        def writeback(out_hbm):
            r0 = tid * RPT
            pltpu.sync_copy(acc.at[pl.ds(r0, RPT)],
                            out_hbm.at[pl.ds(cid * N_ACC + r0, RPT)])

        zero_acc()
        plsc.subcore_barrier()
        accumulate(ujr_hbm, ujc_hbm, ujv_hbm, bot_hbm)
        plsc.subcore_barrier()
        writeback(out_u_hbm)

        zero_acc()
        plsc.subcore_barrier()
        accumulate(ijr_hbm, ijc_hbm, ijv_hbm, bot_hbm)
        plsc.subcore_barrier()
        writeback(out_t_hbm)

        zero_acc()
        plsc.subcore_barrier()
        accumulate(ujc_hbm, ujr_hbm, ujv_hbm, usr_hbm)
        accumulate(ijc_hbm, ijr_hbm, ijv_hbm, top_hbm)
        plsc.subcore_barrier()
        writeback(out_p_hbm)

    out_sds = jax.ShapeDtypeStruct((2 * N_ACC, DH), f32)
    run = pl.kernel(
        body,
        out_type=(out_sds, out_sds, out_sds),
        mesh=mesh,
        compiler_params=pltpu.CompilerParams(use_tc_tiling_on_sc=False),
        scratch_types=(
            pltpu.VMEM_SHARED((N_ACC, DH), f32),
            pltpu.VMEM((PH, C), jnp.int32),
            pltpu.VMEM((PH, C), jnp.int32),
            pltpu.VMEM((PH, C), f32),
            pltpu.VMEM((C, DH), f32),
            pltpu.VMEM((C, DH), f32),
            pltpu.VMEM((C, DH), f32),
            pltpu.SemaphoreType.DMA,
            pltpu.SemaphoreType.DMA,
            pltpu.SemaphoreType.DMA,
            pltpu.SemaphoreType.DMA,
            pltpu.SemaphoreType.DMA,
            pltpu.SemaphoreType.DMA,
        ),
    )
    return run(bot, usr, top, ujr, ujc, ujv, ijr, ijc, ijv)


def kernel(adj_UJ_indices, adj_UJ_values, adj_IJ_indices, adj_IJ_values,
           top_embs, pos_bottoms_embs, all_users_embs):
    i32 = jnp.int32

    def pad_idx(x):
        return jnp.pad(x.astype(i32), (0, E_PAD - E)).reshape(E_PAD // C, C)

    def pad_val(x):
        return jnp.pad(x, (0, E_PAD - E)).reshape(E_PAD // C, C)

    ujr = pad_idx(adj_UJ_indices[0])
    ujc = pad_idx(adj_UJ_indices[1])
    ijr = pad_idx(adj_IJ_indices[0])
    ijc = pad_idx(adj_IJ_indices[1])
    ujv = pad_val(adj_UJ_values)
    ijv = pad_val(adj_IJ_values)

    def stack_halves(x):
        return jnp.concatenate([x[:, :DH], x[:, DH:]], axis=0)

    bot = stack_halves(pos_bottoms_embs)
    usr = stack_halves(all_users_embs)
    top = stack_halves(top_embs)

    out_u, out_t, out_p = _sc_lightgcn(bot, usr, top, ujr, ujc, ujv,
                                       ijr, ijc, ijv)

    def unstack(o):
        return jnp.concatenate([o[:N_ROWS], o[N_ACC:N_ACC + N_ROWS]], axis=1)

    return (unstack(out_u), unstack(out_t), unstack(out_p))
